# baseline (speedup 1.0000x reference)
import functools

import jax
import jax.numpy as jnp
from jax import lax
from jax.experimental import pallas as pl
from jax.experimental.pallas import tpu as pltpu
from jax.experimental.pallas import tpu_sc as plsc

N = 10000
E = 320000
D_IN = 128
H = 32
B = 16
A_DIM = 8
R_DIM = 4
NET = 256
W128 = 128
DW = 8

NC = 2
NS = 16
NW = NC * NS
CH = 128
NCHUNK = 80
EPT = NCHUNK * CH
EPAD = NW * EPT
NP = 10112
RZ = NP // NS
NLAST = N - (NS - 1) * RZ

NBUF = 4

RB = 2000
NG = N // RB
EROWS = E // CH
SROWS = NW * NCHUNK
ERB = EROWS // NG
SRB = SROWS // NG

_MESH = plsc.VectorSubcoreMesh(
    core_axis_name="c", subcore_axis_name="s", num_cores=NC, num_subcores=NS)


def _sc_edge_body(with_deg, src_hbm, dst_hbm, g_hbm, zeros_hbm, *rest):
    if with_deg:
        (ones_hbm, out_hbm,
         acc_sp, deg_sp, g_sp, src_v, dst_v, rows_v, ones_v, sems, dsem) = rest
    else:
        (out_hbm, acc_sp, g_sp, src_v, dst_v, rows_v, sems) = rest
    c = lax.axis_index("c")
    s = lax.axis_index("s")
    wid = c * NS + s

    gsems, ssems = sems
    psem = gsems.at[NBUF - 1]
    pro = [
        pltpu.async_copy(zeros_hbm.at[pl.ds(s * RZ, RZ)],
                         acc_sp.at[pl.ds(s * RZ, RZ)], psem),
        pltpu.async_copy(src_hbm.at[pl.ds(wid * NCHUNK, NCHUNK)], src_v, psem),
        pltpu.async_copy(dst_hbm.at[pl.ds(wid * NCHUNK, NCHUNK)], dst_v, psem),
    ]

    @pl.when(s < NS - 1)
    def _stage():
        pltpu.async_copy(g_hbm.at[pl.ds(s * RZ, RZ), pl.ds(0, H)],
                         g_sp.at[pl.ds(s * RZ, RZ)], psem)

    @pl.when(s == NS - 1)
    def _stage_last():
        pltpu.async_copy(g_hbm.at[pl.ds((NS - 1) * RZ, NLAST), pl.ds(0, H)],
                         g_sp.at[pl.ds((NS - 1) * RZ, NLAST)], psem)

    if with_deg:
        pro.append(pltpu.async_copy(zeros_hbm.at[pl.ds(s * RZ, RZ), pl.ds(0, DW)],
                                    deg_sp.at[pl.ds(s * RZ, RZ)], psem))
        pro.append(pltpu.async_copy(ones_hbm, ones_v, psem))
    for d in pro:
        d.wait()

    @pl.when(s < NS - 1)
    def _stage_wait():
        pltpu.make_async_copy(g_hbm.at[pl.ds(s * RZ, RZ), pl.ds(0, H)],
                              g_sp.at[pl.ds(s * RZ, RZ)], psem).wait()

    @pl.when(s == NS - 1)
    def _stage_wait_last():
        pltpu.make_async_copy(
            g_hbm.at[pl.ds((NS - 1) * RZ, NLAST), pl.ds(0, H)],
            g_sp.at[pl.ds((NS - 1) * RZ, NLAST)], psem).wait()
    plsc.subcore_barrier()

    pltpu.async_copy(g_sp.at[src_v.at[0]], rows_v.at[0], gsems.at[0])
    pltpu.async_copy(g_sp.at[src_v.at[1]], rows_v.at[1], gsems.at[1])

    def outer(jj, carry):
        for b in range(NBUF):
            i = jj * NBUF + b
            bn = (b + 2) % NBUF

            @pl.when(jnp.logical_and(i >= 2, i + 2 < NCHUNK))
            def _drain():
                pltpu.make_async_copy(
                    rows_v.at[bn], acc_sp.at[dst_v.at[i]], ssems.at[bn]).wait()

            @pl.when(i + 2 < NCHUNK)
            def _fire():
                pltpu.async_copy(
                    g_sp.at[src_v.at[i + 2]], rows_v.at[bn], gsems.at[bn])

            pltpu.make_async_copy(
                g_sp.at[src_v.at[i]], rows_v.at[b], gsems.at[b]).wait()
            pltpu.async_copy(rows_v.at[b], acc_sp.at[dst_v.at[i]],
                             ssems.at[b], add=True)
            if with_deg:
                pltpu.async_copy(ones_v, deg_sp.at[dst_v.at[i]],
                                 dsem, add=True)
        return carry

    lax.fori_loop(0, NCHUNK // NBUF, outer, 0)

    for b in range(NBUF):
        pltpu.make_async_copy(
            rows_v.at[b], acc_sp.at[dst_v.at[0]], ssems.at[b]).wait()
    if with_deg:
        def drain_deg(i, carry):
            pltpu.make_async_copy(
                ones_v, deg_sp.at[dst_v.at[0]], dsem).wait()
            return carry
        lax.fori_loop(0, NCHUNK, drain_deg, 0)
    plsc.subcore_barrier()

    for cc in range(NC):
        @pl.when(c == cc)
        def _copy_out(cc=cc):
            pltpu.sync_copy(acc_sp.at[pl.ds(s * RZ, RZ)],
                            out_hbm.at[pl.ds(s * RZ, RZ), pl.ds(cc * 64, H)])
            if with_deg:
                pltpu.sync_copy(
                    deg_sp.at[pl.ds(s * RZ, RZ)],
                    out_hbm.at[pl.ds(s * RZ, RZ), pl.ds(cc * 64 + H, DW)])


_sc_scatter_deg = pl.kernel(
    functools.partial(_sc_edge_body, True),
    out_type=jax.ShapeDtypeStruct((NP, W128), jnp.float32),
    mesh=_MESH,
    compiler_params=pltpu.CompilerParams(use_tc_tiling_on_sc=False),
    scratch_types=[
        pltpu.VMEM_SHARED((NP, H), jnp.float32),
        pltpu.VMEM_SHARED((NP, DW), jnp.float32),
        pltpu.VMEM_SHARED((N, H), jnp.float32),
        pltpu.VMEM((NCHUNK, CH), jnp.int32),
        pltpu.VMEM((NCHUNK, CH), jnp.int32),
        pltpu.VMEM((NBUF, CH, H), jnp.float32),
        pltpu.VMEM((CH, DW), jnp.float32),
        (pltpu.SemaphoreType.DMA((NBUF,)), pltpu.SemaphoreType.DMA((NBUF,))),
        pltpu.SemaphoreType.DMA,
    ],
)

_sc_scatter = pl.kernel(
    functools.partial(_sc_edge_body, False),
    out_type=jax.ShapeDtypeStruct((NP, W128), jnp.float32),
    mesh=_MESH,
    compiler_params=pltpu.CompilerParams(use_tc_tiling_on_sc=False),
    scratch_types=[
        pltpu.VMEM_SHARED((NP, H), jnp.float32),
        pltpu.VMEM_SHARED((N, H), jnp.float32),
        pltpu.VMEM((NCHUNK, CH), jnp.int32),
        pltpu.VMEM((NCHUNK, CH), jnp.int32),
        pltpu.VMEM((NBUF, CH, H), jnp.float32),
        (pltpu.SemaphoreType.DMA((NBUF,)), pltpu.SemaphoreType.DMA((NBUF,))),
    ],
)


def _col32():
    return (lax.broadcasted_iota(jnp.int32, (1, W128), 1) == H).astype(
        jnp.float32)


def _tc1_body(x_ref, wn_ref, es_ref, ed_ref, g_ref, so_ref, do_ref):
    g_ref[...] = jnp.dot(x_ref[...], wn_ref[...],
                         preferred_element_type=jnp.float32)
    row = (lax.broadcasted_iota(jnp.int32, (SRB, CH), 0)
           + pl.program_id(0) * SRB)
    lane = lax.broadcasted_iota(jnp.int32, (SRB, CH), 1)
    valid = row < EROWS
    so_ref[...] = jnp.where(valid, es_ref[...], 0)
    do_ref[...] = jnp.where(valid, ed_ref[...], N + (lane % 16))


def _tc2_body(x_ref, sd_ref, ws0_ref, b_ref, wsn_ref, wnn_ref, hs_ref, g_ref):
    ssum = sd_ref[:, :H] + sd_ref[:, 64:64 + H]
    deg = sd_ref[:, H:H + 1] + sd_ref[:, 96:97]
    dinv = 1.0 / jnp.maximum(deg, 1.0)
    xs = jnp.dot(x_ref[...], ws0_ref[...], preferred_element_type=jnp.float32)
    h = jnp.maximum(xs + ssum * dinv + b_ref[...], 0.0)
    hs_ref[...] = (jnp.dot(h, wsn_ref[...], preferred_element_type=jnp.float32)
                   + dinv * _col32())
    g_ref[...] = jnp.dot(h, wnn_ref[...], preferred_element_type=jnp.float32)


def _tc3_body(hs_ref, sp_ref, b_ref, wsn_ref, wnn_ref, hs2_ref, g2_ref):
    ssum = sp_ref[:, :H] + sp_ref[:, 64:64 + H]
    dinv = hs_ref[:, H:H + 1]
    h = jnp.maximum(hs_ref[:, :H] + ssum * dinv + b_ref[...], 0.0)
    hs2_ref[...] = (jnp.dot(h, wsn_ref[...], preferred_element_type=jnp.float32)
                    + dinv * _col32())
    g2_ref[...] = jnp.dot(h, wnn_ref[...], preferred_element_type=jnp.float32)


def _tc4_body(hs_ref, sp_ref, b_ref, mask_ref, a2_ref,
              wc1h_ref, wc1a_ref, bc1_ref, wc2_ref, bc2_ref, wc3_ref, bc3_ref,
              q_ref, acc_ref, cnt_ref):
    i = pl.program_id(0)
    ssum = sp_ref[:, :H] + sp_ref[:, 64:64 + H]
    dinv = hs_ref[:, H:H + 1]
    h3 = jnp.maximum(hs_ref[:, :H] + ssum * dinv + b_ref[...], 0.0)
    m = mask_ref[:, :B]
    dn = (((0,), (0,)), ((), ()))
    pacc = lax.dot_general(m, h3, dn,
                           preferred_element_type=jnp.float32)
    pcnt = lax.dot_general(m, jnp.ones((RB, 1), jnp.float32), dn,
                           preferred_element_type=jnp.float32)

    @pl.when(i == 0)
    def _init():
        acc_ref[...] = pacc
        cnt_ref[...] = pcnt

    @pl.when(i > 0)
    def _accum():
        acc_ref[...] += pacc
        cnt_ref[...] += pcnt

    @pl.when(i == NG - 1)
    def _finish():
        nf = acc_ref[...] / jnp.maximum(cnt_ref[...], 1.0)
        z = jnp.dot(nf, wc1h_ref[...], preferred_element_type=jnp.float32)
        z = z + jnp.dot(a2_ref[...], wc1a_ref[...],
                        preferred_element_type=jnp.float32) + bc1_ref[...]
        z = jnp.maximum(z, 0.0)
        z = jnp.maximum(jnp.dot(z, wc2_ref[...],
                                preferred_element_type=jnp.float32)
                        + bc2_ref[...], 0.0)
        q_ref[...] = jnp.dot(z, wc3_ref[...],
                             preferred_element_type=jnp.float32) + bc3_ref[...]


def _row_spec(cols):
    return pl.BlockSpec((RB, cols), lambda i: (i, 0))


def _full_spec(shape):
    nd = len(shape)
    return pl.BlockSpec(shape, lambda i, _nd=nd: (0,) * _nd)


def _part_spec():
    return pl.BlockSpec((RB, W128), lambda i: (i, 0))


_tc1 = pl.pallas_call(
    _tc1_body,
    grid=(NG,),
    in_specs=[_row_spec(D_IN), _full_spec((D_IN, W128)),
              pl.BlockSpec((SRB, CH), lambda i: (i, 0)),
              pl.BlockSpec((SRB, CH), lambda i: (i, 0))],
    out_specs=(_row_spec(W128),
               pl.BlockSpec((SRB, CH), lambda i: (i, 0)),
               pl.BlockSpec((SRB, CH), lambda i: (i, 0))),
    out_shape=(jax.ShapeDtypeStruct((N, W128), jnp.float32),
               jax.ShapeDtypeStruct((SROWS, CH), jnp.int32),
               jax.ShapeDtypeStruct((SROWS, CH), jnp.int32)),
    compiler_params=pltpu.CompilerParams(
        dimension_semantics=("parallel",)),
)

_tc2 = pl.pallas_call(
    _tc2_body,
    grid=(NG,),
    in_specs=[_row_spec(D_IN), _part_spec(), _full_spec((D_IN, H)),
              _full_spec((1, H)),
              _full_spec((H, W128)), _full_spec((H, W128))],
    out_specs=(_row_spec(W128), _row_spec(W128)),
    out_shape=(jax.ShapeDtypeStruct((N, W128), jnp.float32),
               jax.ShapeDtypeStruct((N, W128), jnp.float32)),
    compiler_params=pltpu.CompilerParams(
        dimension_semantics=("parallel",)),
)

_tc3 = pl.pallas_call(
    _tc3_body,
    grid=(NG,),
    in_specs=[_row_spec(W128), _part_spec(), _full_spec((1, H)),
              _full_spec((H, W128)), _full_spec((H, W128))],
    out_specs=(_row_spec(W128), _row_spec(W128)),
    out_shape=(jax.ShapeDtypeStruct((N, W128), jnp.float32),
               jax.ShapeDtypeStruct((N, W128), jnp.float32)),
    compiler_params=pltpu.CompilerParams(
        dimension_semantics=("parallel",)),
)

_tc4 = pl.pallas_call(
    _tc4_body,
    grid=(NG,),
    in_specs=[_row_spec(W128), _part_spec(), _full_spec((1, H)),
              _row_spec(W128), _full_spec((B, A_DIM)),
              _full_spec((H, NET)), _full_spec((A_DIM, NET)),
              _full_spec((1, NET)), _full_spec((NET, NET)),
              _full_spec((1, NET)), _full_spec((NET, R_DIM)),
              _full_spec((1, R_DIM))],
    out_specs=_full_spec((B, R_DIM)),
    out_shape=jax.ShapeDtypeStruct((B, R_DIM), jnp.float32),
    scratch_shapes=[pltpu.VMEM((B, H), jnp.float32),
                    pltpu.VMEM((B, 1), jnp.float32)],
    compiler_params=pltpu.CompilerParams(
        dimension_semantics=("arbitrary",)),
)


def _pad128(w):
    return jnp.pad(w, ((0, 0), (0, W128 - w.shape[1])))


def kernel(x, edge_index, node_graph_ids, a,
           Wself0, Wneigh0, b0, Wself1, Wneigh1, b1, Wself2, Wneigh2, b2,
           Wc1, bc1, Wc2, bc2, Wc3, bc3):
    zeros = jnp.zeros((NP, H), jnp.float32)
    ones = jnp.ones((CH, DW), jnp.float32)

    g0, src, dst = _tc1(x, _pad128(Wneigh0),
                        edge_index[0].reshape(EROWS, CH),
                        edge_index[1].reshape(EROWS, CH))
    sd0 = _sc_scatter_deg(src, dst, g0, zeros, ones)
    h1s, g1 = _tc2(x, sd0, Wself0, b0.reshape(1, H),
                   _pad128(Wself1), _pad128(Wneigh1))
    s1 = _sc_scatter(src, dst, g1, zeros)
    h2s, g2 = _tc3(h1s, s1, b1.reshape(1, H),
                   _pad128(Wself2), _pad128(Wneigh2))
    s2 = _sc_scatter(src, dst, g2, zeros)

    mask = (node_graph_ids[:, None]
            == jnp.arange(W128, dtype=jnp.int32)[None, :]).astype(jnp.float32)
    a2 = jnp.squeeze(a, -1)
    q = _tc4(h2s, s2, b2.reshape(1, H), mask, a2,
             Wc1[:H], Wc1[H:], bc1.reshape(1, NET), Wc2,
             bc2.reshape(1, NET), Wc3, bc3.reshape(1, R_DIM))
    return q


# speedup vs baseline: 1.0004x; 1.0004x over previous
import functools

import jax
import jax.numpy as jnp
from jax import lax
from jax.experimental import pallas as pl
from jax.experimental.pallas import tpu as pltpu
from jax.experimental.pallas import tpu_sc as plsc

N = 10000
E = 320000
D_IN = 128
H = 32
B = 16
A_DIM = 8
R_DIM = 4
NET = 256
W128 = 128
DW = 8

NC = 2
NS = 16
NW = NC * NS
CH = 128
NCHUNK = 80
EPT = NCHUNK * CH
EPAD = NW * EPT
NP = 10112
RZ = NP // NS
NLAST = N - (NS - 1) * RZ

NBUF = 8
LA = 3

RB = 2000
NG = N // RB
EROWS = E // CH
SROWS = NW * NCHUNK
ERB = EROWS // NG
SRB = SROWS // NG

_MESH = plsc.VectorSubcoreMesh(
    core_axis_name="c", subcore_axis_name="s", num_cores=NC, num_subcores=NS)


def _sc_edge_body(with_deg, src_hbm, dst_hbm, g_hbm, zeros_hbm, *rest):
    if with_deg:
        (ones_hbm, out_hbm,
         acc_sp, deg_sp, g_sp, src_v, dst_v, rows_v, ones_v, sems, dsem) = rest
    else:
        (out_hbm, acc_sp, g_sp, src_v, dst_v, rows_v, sems) = rest
    c = lax.axis_index("c")
    s = lax.axis_index("s")
    wid = c * NS + s

    gsems, ssems = sems
    psem = gsems.at[NBUF - 1]
    pro = [
        pltpu.async_copy(zeros_hbm.at[pl.ds(s * RZ, RZ)],
                         acc_sp.at[pl.ds(s * RZ, RZ)], psem),
        pltpu.async_copy(src_hbm.at[pl.ds(wid * NCHUNK, NCHUNK)], src_v, psem),
        pltpu.async_copy(dst_hbm.at[pl.ds(wid * NCHUNK, NCHUNK)], dst_v, psem),
    ]

    @pl.when(s < NS - 1)
    def _stage():
        pltpu.async_copy(g_hbm.at[pl.ds(s * RZ, RZ), pl.ds(0, H)],
                         g_sp.at[pl.ds(s * RZ, RZ)], psem)

    @pl.when(s == NS - 1)
    def _stage_last():
        pltpu.async_copy(g_hbm.at[pl.ds((NS - 1) * RZ, NLAST), pl.ds(0, H)],
                         g_sp.at[pl.ds((NS - 1) * RZ, NLAST)], psem)

    if with_deg:
        pro.append(pltpu.async_copy(zeros_hbm.at[pl.ds(s * RZ, RZ), pl.ds(0, DW)],
                                    deg_sp.at[pl.ds(s * RZ, RZ)], psem))
        pro.append(pltpu.async_copy(ones_hbm, ones_v, psem))
    for d in pro:
        d.wait()

    @pl.when(s < NS - 1)
    def _stage_wait():
        pltpu.make_async_copy(g_hbm.at[pl.ds(s * RZ, RZ), pl.ds(0, H)],
                              g_sp.at[pl.ds(s * RZ, RZ)], psem).wait()

    @pl.when(s == NS - 1)
    def _stage_wait_last():
        pltpu.make_async_copy(
            g_hbm.at[pl.ds((NS - 1) * RZ, NLAST), pl.ds(0, H)],
            g_sp.at[pl.ds((NS - 1) * RZ, NLAST)], psem).wait()
    plsc.subcore_barrier()

    for p in range(LA):
        pltpu.async_copy(g_sp.at[src_v.at[p]], rows_v.at[p], gsems.at[p])

    def outer(jj, carry):
        for b in range(NBUF):
            i = jj * NBUF + b
            bn = (b + LA) % NBUF

            @pl.when(jnp.logical_and(i >= NBUF - LA, i + LA < NCHUNK))
            def _drain():
                pltpu.make_async_copy(
                    rows_v.at[bn], acc_sp.at[dst_v.at[i]], ssems.at[bn]).wait()

            @pl.when(i + LA < NCHUNK)
            def _fire():
                pltpu.async_copy(
                    g_sp.at[src_v.at[i + LA]], rows_v.at[bn], gsems.at[bn])

            pltpu.make_async_copy(
                g_sp.at[src_v.at[i]], rows_v.at[b], gsems.at[b]).wait()
            pltpu.async_copy(rows_v.at[b], acc_sp.at[dst_v.at[i]],
                             ssems.at[b], add=True)
            if with_deg:
                pltpu.async_copy(ones_v, deg_sp.at[dst_v.at[i]],
                                 dsem, add=True)
        return carry

    lax.fori_loop(0, NCHUNK // NBUF, outer, 0)

    for b in range(NBUF):
        pltpu.make_async_copy(
            rows_v.at[b], acc_sp.at[dst_v.at[0]], ssems.at[b]).wait()
    if with_deg:
        def drain_deg(i, carry):
            pltpu.make_async_copy(
                ones_v, deg_sp.at[dst_v.at[0]], dsem).wait()
            return carry
        lax.fori_loop(0, NCHUNK, drain_deg, 0)
    plsc.subcore_barrier()

    for cc in range(NC):
        @pl.when(c == cc)
        def _copy_out(cc=cc):
            pltpu.sync_copy(acc_sp.at[pl.ds(s * RZ, RZ)],
                            out_hbm.at[pl.ds(s * RZ, RZ), pl.ds(cc * 64, H)])
            if with_deg:
                pltpu.sync_copy(
                    deg_sp.at[pl.ds(s * RZ, RZ)],
                    out_hbm.at[pl.ds(s * RZ, RZ), pl.ds(cc * 64 + H, DW)])


_sc_scatter_deg = pl.kernel(
    functools.partial(_sc_edge_body, True),
    out_type=jax.ShapeDtypeStruct((NP, W128), jnp.float32),
    mesh=_MESH,
    compiler_params=pltpu.CompilerParams(use_tc_tiling_on_sc=False),
    scratch_types=[
        pltpu.VMEM_SHARED((NP, H), jnp.float32),
        pltpu.VMEM_SHARED((NP, DW), jnp.float32),
        pltpu.VMEM_SHARED((N, H), jnp.float32),
        pltpu.VMEM((NCHUNK, CH), jnp.int32),
        pltpu.VMEM((NCHUNK, CH), jnp.int32),
        pltpu.VMEM((NBUF, CH, H), jnp.float32),
        pltpu.VMEM((CH, DW), jnp.float32),
        (pltpu.SemaphoreType.DMA((NBUF,)), pltpu.SemaphoreType.DMA((NBUF,))),
        pltpu.SemaphoreType.DMA,
    ],
)

_sc_scatter = pl.kernel(
    functools.partial(_sc_edge_body, False),
    out_type=jax.ShapeDtypeStruct((NP, W128), jnp.float32),
    mesh=_MESH,
    compiler_params=pltpu.CompilerParams(use_tc_tiling_on_sc=False),
    scratch_types=[
        pltpu.VMEM_SHARED((NP, H), jnp.float32),
        pltpu.VMEM_SHARED((N, H), jnp.float32),
        pltpu.VMEM((NCHUNK, CH), jnp.int32),
        pltpu.VMEM((NCHUNK, CH), jnp.int32),
        pltpu.VMEM((NBUF, CH, H), jnp.float32),
        (pltpu.SemaphoreType.DMA((NBUF,)), pltpu.SemaphoreType.DMA((NBUF,))),
    ],
)


def _col32():
    return (lax.broadcasted_iota(jnp.int32, (1, W128), 1) == H).astype(
        jnp.float32)


def _tc1_body(x_ref, wn_ref, es_ref, ed_ref, g_ref, so_ref, do_ref):
    g_ref[...] = jnp.dot(x_ref[...], wn_ref[...],
                         preferred_element_type=jnp.float32)
    row = (lax.broadcasted_iota(jnp.int32, (SRB, CH), 0)
           + pl.program_id(0) * SRB)
    lane = lax.broadcasted_iota(jnp.int32, (SRB, CH), 1)
    valid = row < EROWS
    so_ref[...] = jnp.where(valid, es_ref[...], 0)
    do_ref[...] = jnp.where(valid, ed_ref[...], N + (lane % 16))


def _tc2_body(x_ref, sd_ref, ws0_ref, b_ref, wsn_ref, wnn_ref, hs_ref, g_ref):
    ssum = sd_ref[:, :H] + sd_ref[:, 64:64 + H]
    deg = sd_ref[:, H:H + 1] + sd_ref[:, 96:97]
    dinv = 1.0 / jnp.maximum(deg, 1.0)
    xs = jnp.dot(x_ref[...], ws0_ref[...], preferred_element_type=jnp.float32)
    h = jnp.maximum(xs + ssum * dinv + b_ref[...], 0.0)
    hs_ref[...] = (jnp.dot(h, wsn_ref[...], preferred_element_type=jnp.float32)
                   + dinv * _col32())
    g_ref[...] = jnp.dot(h, wnn_ref[...], preferred_element_type=jnp.float32)


def _tc3_body(hs_ref, sp_ref, b_ref, wsn_ref, wnn_ref, hs2_ref, g2_ref):
    ssum = sp_ref[:, :H] + sp_ref[:, 64:64 + H]
    dinv = hs_ref[:, H:H + 1]
    h = jnp.maximum(hs_ref[:, :H] + ssum * dinv + b_ref[...], 0.0)
    hs2_ref[...] = (jnp.dot(h, wsn_ref[...], preferred_element_type=jnp.float32)
                    + dinv * _col32())
    g2_ref[...] = jnp.dot(h, wnn_ref[...], preferred_element_type=jnp.float32)


def _tc4_body(hs_ref, sp_ref, b_ref, mask_ref, a2_ref,
              wc1h_ref, wc1a_ref, bc1_ref, wc2_ref, bc2_ref, wc3_ref, bc3_ref,
              q_ref, acc_ref, cnt_ref):
    i = pl.program_id(0)
    ssum = sp_ref[:, :H] + sp_ref[:, 64:64 + H]
    dinv = hs_ref[:, H:H + 1]
    h3 = jnp.maximum(hs_ref[:, :H] + ssum * dinv + b_ref[...], 0.0)
    m = mask_ref[:, :B]
    dn = (((0,), (0,)), ((), ()))
    pacc = lax.dot_general(m, h3, dn,
                           preferred_element_type=jnp.float32)
    pcnt = lax.dot_general(m, jnp.ones((RB, 1), jnp.float32), dn,
                           preferred_element_type=jnp.float32)

    @pl.when(i == 0)
    def _init():
        acc_ref[...] = pacc
        cnt_ref[...] = pcnt

    @pl.when(i > 0)
    def _accum():
        acc_ref[...] += pacc
        cnt_ref[...] += pcnt

    @pl.when(i == NG - 1)
    def _finish():
        nf = acc_ref[...] / jnp.maximum(cnt_ref[...], 1.0)
        z = jnp.dot(nf, wc1h_ref[...], preferred_element_type=jnp.float32)
        z = z + jnp.dot(a2_ref[...], wc1a_ref[...],
                        preferred_element_type=jnp.float32) + bc1_ref[...]
        z = jnp.maximum(z, 0.0)
        z = jnp.maximum(jnp.dot(z, wc2_ref[...],
                                preferred_element_type=jnp.float32)
                        + bc2_ref[...], 0.0)
        q_ref[...] = jnp.dot(z, wc3_ref[...],
                             preferred_element_type=jnp.float32) + bc3_ref[...]


def _row_spec(cols):
    return pl.BlockSpec((RB, cols), lambda i: (i, 0))


def _full_spec(shape):
    nd = len(shape)
    return pl.BlockSpec(shape, lambda i, _nd=nd: (0,) * _nd)


def _part_spec():
    return pl.BlockSpec((RB, W128), lambda i: (i, 0))


_tc1 = pl.pallas_call(
    _tc1_body,
    grid=(NG,),
    in_specs=[_row_spec(D_IN), _full_spec((D_IN, W128)),
              pl.BlockSpec((SRB, CH), lambda i: (i, 0)),
              pl.BlockSpec((SRB, CH), lambda i: (i, 0))],
    out_specs=(_row_spec(W128),
               pl.BlockSpec((SRB, CH), lambda i: (i, 0)),
               pl.BlockSpec((SRB, CH), lambda i: (i, 0))),
    out_shape=(jax.ShapeDtypeStruct((N, W128), jnp.float32),
               jax.ShapeDtypeStruct((SROWS, CH), jnp.int32),
               jax.ShapeDtypeStruct((SROWS, CH), jnp.int32)),
    compiler_params=pltpu.CompilerParams(
        dimension_semantics=("parallel",)),
)

_tc2 = pl.pallas_call(
    _tc2_body,
    grid=(NG,),
    in_specs=[_row_spec(D_IN), _part_spec(), _full_spec((D_IN, H)),
              _full_spec((1, H)),
              _full_spec((H, W128)), _full_spec((H, W128))],
    out_specs=(_row_spec(W128), _row_spec(W128)),
    out_shape=(jax.ShapeDtypeStruct((N, W128), jnp.float32),
               jax.ShapeDtypeStruct((N, W128), jnp.float32)),
    compiler_params=pltpu.CompilerParams(
        dimension_semantics=("parallel",)),
)

_tc3 = pl.pallas_call(
    _tc3_body,
    grid=(NG,),
    in_specs=[_row_spec(W128), _part_spec(), _full_spec((1, H)),
              _full_spec((H, W128)), _full_spec((H, W128))],
    out_specs=(_row_spec(W128), _row_spec(W128)),
    out_shape=(jax.ShapeDtypeStruct((N, W128), jnp.float32),
               jax.ShapeDtypeStruct((N, W128), jnp.float32)),
    compiler_params=pltpu.CompilerParams(
        dimension_semantics=("parallel",)),
)

_tc4 = pl.pallas_call(
    _tc4_body,
    grid=(NG,),
    in_specs=[_row_spec(W128), _part_spec(), _full_spec((1, H)),
              _row_spec(W128), _full_spec((B, A_DIM)),
              _full_spec((H, NET)), _full_spec((A_DIM, NET)),
              _full_spec((1, NET)), _full_spec((NET, NET)),
              _full_spec((1, NET)), _full_spec((NET, R_DIM)),
              _full_spec((1, R_DIM))],
    out_specs=_full_spec((B, R_DIM)),
    out_shape=jax.ShapeDtypeStruct((B, R_DIM), jnp.float32),
    scratch_shapes=[pltpu.VMEM((B, H), jnp.float32),
                    pltpu.VMEM((B, 1), jnp.float32)],
    compiler_params=pltpu.CompilerParams(
        dimension_semantics=("arbitrary",)),
)


def _pad128(w):
    return jnp.pad(w, ((0, 0), (0, W128 - w.shape[1])))


def kernel(x, edge_index, node_graph_ids, a,
           Wself0, Wneigh0, b0, Wself1, Wneigh1, b1, Wself2, Wneigh2, b2,
           Wc1, bc1, Wc2, bc2, Wc3, bc3):
    zeros = jnp.zeros((NP, H), jnp.float32)
    ones = jnp.ones((CH, DW), jnp.float32)

    g0, src, dst = _tc1(x, _pad128(Wneigh0),
                        edge_index[0].reshape(EROWS, CH),
                        edge_index[1].reshape(EROWS, CH))
    sd0 = _sc_scatter_deg(src, dst, g0, zeros, ones)
    h1s, g1 = _tc2(x, sd0, Wself0, b0.reshape(1, H),
                   _pad128(Wself1), _pad128(Wneigh1))
    s1 = _sc_scatter(src, dst, g1, zeros)
    h2s, g2 = _tc3(h1s, s1, b1.reshape(1, H),
                   _pad128(Wself2), _pad128(Wneigh2))
    s2 = _sc_scatter(src, dst, g2, zeros)

    mask = (node_graph_ids[:, None]
            == jnp.arange(W128, dtype=jnp.int32)[None, :]).astype(jnp.float32)
    a2 = jnp.squeeze(a, -1)
    q = _tc4(h2s, s2, b2.reshape(1, H), mask, a2,
             Wc1[:H], Wc1[H:], bc1.reshape(1, NET), Wc2,
             bc2.reshape(1, NET), Wc3, bc3.reshape(1, R_DIM))
    return q
